# Initial kernel scaffold; baseline (speedup 1.0000x reference)
import jax
import jax.numpy as jnp
from jax.experimental import pallas as pl


def kernel(x, edge_index, edge_attr, u, batch, W1, b1, W2, b2):
    raise NotImplementedError("write your pallas kernel here")


# speedup vs baseline: 6.7136x; 6.7136x over previous
import functools

import jax
import jax.numpy as jnp
from jax import lax
from jax.experimental import pallas as pl
from jax.experimental.pallas import tpu as pltpu
from jax.experimental.pallas import tpu_sc as plsc

N_NODES = 10000
N_EDGES = 320000
D = 128
NC = 2
NS = 16
NW = NC * NS
EPW = N_EDGES // NW
CH = 80
NCH = EPW // CH
N_PAD = 10240
ROWS_PT = N_PAD // NS

_mesh = plsc.VectorSubcoreMesh(core_axis_name="c", subcore_axis_name="s")


@functools.partial(
    pl.kernel,
    out_type=jax.ShapeDtypeStruct((NC, N_PAD, D), jnp.float32),
    mesh=_mesh,
    scratch_types=[
        pltpu.VMEM((NCH, CH), jnp.int32),
        pltpu.VMEM((CH, D), jnp.float32),
        pltpu.VMEM((CH, D), jnp.float32),
        pltpu.VMEM_SHARED((N_PAD, D), jnp.float32),
        pltpu.SemaphoreType.DMA,
        pltpu.SemaphoreType.DMA,
    ],
)
def _scatter_sc(row_hbm, ea_hbm, zeros_hbm, out_hbm,
                idx_v, buf0, buf1, agg_s, rs0, rs1):
    cid = lax.axis_index("c")
    sid = lax.axis_index("s")
    wid = sid * NC + cid
    r0 = sid * ROWS_PT

    pltpu.sync_copy(zeros_hbm.at[pl.ds(r0, ROWS_PT)],
                    agg_s.at[pl.ds(r0, ROWS_PT)])
    pltpu.sync_copy(row_hbm.at[wid], idx_v)
    plsc.subcore_barrier()

    ebase = wid * EPW
    pltpu.async_copy(ea_hbm.at[pl.ds(ebase, CH)], buf0, rs0)

    def step(i, carry):
        j0 = 2 * i
        j1 = 2 * i + 1
        pltpu.async_copy(ea_hbm.at[pl.ds(ebase + j1 * CH, CH)], buf1, rs1)
        pltpu.make_async_copy(ea_hbm.at[pl.ds(ebase, CH)], buf0, rs0).wait()
        pltpu.sync_copy(buf0, agg_s.at[idx_v.at[j0]], add=True)
        j2 = jnp.minimum(j0 + 2, NCH - 1)
        pltpu.async_copy(ea_hbm.at[pl.ds(ebase + j2 * CH, CH)], buf0, rs0)
        pltpu.make_async_copy(ea_hbm.at[pl.ds(ebase, CH)], buf1, rs1).wait()
        pltpu.sync_copy(buf1, agg_s.at[idx_v.at[j1]], add=True)
        return carry

    lax.fori_loop(0, NCH // 2, step, 0)
    pltpu.make_async_copy(ea_hbm.at[pl.ds(ebase, CH)], buf0, rs0).wait()
    pltpu.sync_copy(buf0, agg_s.at[idx_v.at[NCH - 1]], add=True)

    plsc.subcore_barrier()
    pltpu.sync_copy(agg_s.at[pl.ds(r0, ROWS_PT)],
                    out_hbm.at[cid, pl.ds(r0, ROWS_PT)])


BN = 1000


def _mlp_body(x_ref, p_ref, w1x_ref, w1a_ref, b1_ref, w2_ref, b2_ref, o_ref):
    agg = p_ref[0] + p_ref[1]
    h = jnp.dot(x_ref[...], w1x_ref[...], preferred_element_type=jnp.float32)
    h = h + jnp.dot(agg, w1a_ref[...], preferred_element_type=jnp.float32)
    h = jnp.maximum(h + b1_ref[...], 0.0)
    o_ref[...] = (jnp.dot(h, w2_ref[...], preferred_element_type=jnp.float32)
                  + b2_ref[...])


def _mlp(x, parts, w1x, w1a, b1, w2, b2):
    return pl.pallas_call(
        _mlp_body,
        grid=(N_NODES // BN,),
        in_specs=[
            pl.BlockSpec((BN, D), lambda i: (i, 0)),
            pl.BlockSpec((NC, BN, D), lambda i: (0, i, 0)),
            pl.BlockSpec((D, D), lambda i: (0, 0)),
            pl.BlockSpec((D, D), lambda i: (0, 0)),
            pl.BlockSpec((1, D), lambda i: (0, 0)),
            pl.BlockSpec((D, D), lambda i: (0, 0)),
            pl.BlockSpec((1, D), lambda i: (0, 0)),
        ],
        out_specs=pl.BlockSpec((BN, D), lambda i: (i, 0)),
        out_shape=jax.ShapeDtypeStruct((N_NODES, D), jnp.float32),
    )(x, parts, w1x, w1a, b1, w2, b2)


def kernel(x, edge_index, edge_attr, u, batch, W1, b1, W2, b2):
    row = edge_index[0].astype(jnp.int32).reshape(NW, NCH, CH)
    zeros = jnp.zeros((N_PAD, D), jnp.float32)
    parts = _scatter_sc(row, edge_attr, zeros)
    w1T = W1.T
    return _mlp(x, parts, w1T[:D], w1T[D:], b1.reshape(1, D),
                W2.T, b2.reshape(1, D))


# speedup vs baseline: 7.1906x; 1.0711x over previous
import functools

import jax
import jax.numpy as jnp
from jax import lax
from jax.experimental import pallas as pl
from jax.experimental.pallas import tpu as pltpu
from jax.experimental.pallas import tpu_sc as plsc

N_NODES = 10000
N_EDGES = 320000
D = 128
NC = 2
NS = 16
NW = NC * NS
EPW = N_EDGES // NW
CH = 128
NFC = EPW // CH
TAIL = EPW - NFC * CH
NCH = NFC + 1
N_PAD = 10240
GROW = N_PAD - 1
ROWS_PT = N_PAD // NS

_mesh = plsc.VectorSubcoreMesh(core_axis_name="c", subcore_axis_name="s")


@functools.partial(
    pl.kernel,
    out_type=jax.ShapeDtypeStruct((NC, N_PAD, D), jnp.float32),
    mesh=_mesh,
    scratch_types=[
        pltpu.VMEM((NCH, CH), jnp.int32),
        pltpu.VMEM((CH, D), jnp.float32),
        pltpu.VMEM((CH, D), jnp.float32),
        pltpu.VMEM_SHARED((N_PAD, D), jnp.float32),
        pltpu.SemaphoreType.DMA,
        pltpu.SemaphoreType.DMA,
    ],
)
def _scatter_sc(row_hbm, ea_hbm, zeros_hbm, out_hbm,
                idx_v, buf0, buf1, agg_s, rs0, rs1):
    cid = lax.axis_index("c")
    sid = lax.axis_index("s")
    wid = sid * NC + cid
    r0 = sid * ROWS_PT

    pltpu.sync_copy(zeros_hbm, agg_s.at[pl.ds(r0, ROWS_PT)])
    pltpu.sync_copy(row_hbm.at[wid], idx_v)
    plsc.subcore_barrier()

    ebase = wid * EPW
    pltpu.async_copy(ea_hbm.at[pl.ds(ebase, CH)], buf0, rs0)

    def step(i, carry):
        j0 = 2 * i
        j1 = 2 * i + 1
        pltpu.async_copy(ea_hbm.at[pl.ds(ebase + j1 * CH, CH)], buf1, rs1)
        pltpu.make_async_copy(ea_hbm.at[pl.ds(ebase, CH)], buf0, rs0).wait()
        pltpu.sync_copy(buf0, agg_s.at[idx_v.at[j0]], add=True)
        j2 = jnp.minimum(j0 + 2, NFC - 1)
        pltpu.async_copy(ea_hbm.at[pl.ds(ebase + j2 * CH, CH)], buf0, rs0)
        pltpu.make_async_copy(ea_hbm.at[pl.ds(ebase, CH)], buf1, rs1).wait()
        pltpu.sync_copy(buf1, agg_s.at[idx_v.at[j1]], add=True)
        return carry

    lax.fori_loop(0, NFC // 2, step, 0)
    pltpu.make_async_copy(ea_hbm.at[pl.ds(ebase, CH)], buf0, rs0).wait()
    pltpu.sync_copy(ea_hbm.at[pl.ds(ebase + NFC * CH, TAIL)],
                    buf0.at[pl.ds(0, TAIL)])
    pltpu.sync_copy(buf0, agg_s.at[idx_v.at[NFC]], add=True)

    plsc.subcore_barrier()
    pltpu.sync_copy(agg_s.at[pl.ds(r0, ROWS_PT)],
                    out_hbm.at[cid, pl.ds(r0, ROWS_PT)])


BN = 1000


def _mlp_body(x_ref, p_ref, w1x_ref, w1a_ref, b1_ref, w2_ref, b2_ref, o_ref):
    agg = p_ref[0] + p_ref[1]
    h = jnp.dot(x_ref[...], w1x_ref[...], preferred_element_type=jnp.float32)
    h = h + jnp.dot(agg, w1a_ref[...], preferred_element_type=jnp.float32)
    h = jnp.maximum(h + b1_ref[...], 0.0)
    o_ref[...] = (jnp.dot(h, w2_ref[...], preferred_element_type=jnp.float32)
                  + b2_ref[...])


def _mlp(x, parts, w1x, w1a, b1, w2, b2):
    return pl.pallas_call(
        _mlp_body,
        grid=(N_NODES // BN,),
        in_specs=[
            pl.BlockSpec((BN, D), lambda i: (i, 0)),
            pl.BlockSpec((NC, BN, D), lambda i: (0, i, 0)),
            pl.BlockSpec((D, D), lambda i: (0, 0)),
            pl.BlockSpec((D, D), lambda i: (0, 0)),
            pl.BlockSpec((1, D), lambda i: (0, 0)),
            pl.BlockSpec((D, D), lambda i: (0, 0)),
            pl.BlockSpec((1, D), lambda i: (0, 0)),
        ],
        out_specs=pl.BlockSpec((BN, D), lambda i: (i, 0)),
        out_shape=jax.ShapeDtypeStruct((N_NODES, D), jnp.float32),
    )(x, parts, w1x, w1a, b1, w2, b2)


def kernel(x, edge_index, edge_attr, u, batch, W1, b1, W2, b2):
    row = edge_index[0].astype(jnp.int32).reshape(NW, EPW)
    row = jnp.pad(row, ((0, 0), (0, NCH * CH - EPW)), constant_values=GROW)
    row = row.reshape(NW, NCH, CH)
    zeros = jnp.zeros((ROWS_PT, D), jnp.float32)
    parts = _scatter_sc(row, edge_attr, zeros)
    w1T = W1.T
    return _mlp(x, parts, w1T[:D], w1T[D:], b1.reshape(1, D),
                W2.T, b2.reshape(1, D))


# speedup vs baseline: 7.2690x; 1.0109x over previous
import functools

import jax
import jax.numpy as jnp
from jax import lax
from jax.experimental import pallas as pl
from jax.experimental.pallas import tpu as pltpu
from jax.experimental.pallas import tpu_sc as plsc

N_NODES = 10000
N_EDGES = 320000
D = 128
NC = 2
NS = 16
NW = NC * NS
EPW = N_EDGES // NW
CH = 128
NFC = EPW // CH
TAIL = EPW - NFC * CH
NCH = NFC + 1
IDXW = NCH * CH
N_PAD = 10240
GROW = N_PAD - 1
ROWS_PT = N_PAD // NS

_mesh = plsc.VectorSubcoreMesh(core_axis_name="c", subcore_axis_name="s")


@functools.partial(
    pl.kernel,
    out_type=jax.ShapeDtypeStruct((NC, N_PAD, D), jnp.float32),
    mesh=_mesh,
    scratch_types=[
        pltpu.VMEM((IDXW,), jnp.int32),
        pltpu.VMEM((CH, D), jnp.float32),
        pltpu.VMEM((CH, D), jnp.float32),
        pltpu.VMEM_SHARED((N_PAD, D), jnp.float32),
        pltpu.SemaphoreType.DMA,
        pltpu.SemaphoreType.DMA,
    ],
)
def _scatter_sc(ridx_hbm, ea_hbm, zeros_hbm, out_hbm,
                idx_v, buf0, buf1, agg_s, rs0, rs1):
    cid = lax.axis_index("c")
    sid = lax.axis_index("s")
    wid = sid * NC + cid
    r0 = sid * ROWS_PT
    ebase = wid * EPW

    grow = jnp.full((16,), GROW, jnp.int32)
    for k in range((IDXW - EPW) // 16):
        idx_v[pl.ds(EPW + 16 * k, 16)] = grow
    pltpu.async_copy(ridx_hbm.at[pl.ds(ebase, EPW)], idx_v.at[pl.ds(0, EPW)],
                     rs1)
    pltpu.sync_copy(zeros_hbm, agg_s.at[pl.ds(r0, ROWS_PT)])
    pltpu.make_async_copy(ridx_hbm.at[pl.ds(ebase, EPW)],
                          idx_v.at[pl.ds(0, EPW)], rs1).wait()
    plsc.subcore_barrier()

    pltpu.async_copy(ea_hbm.at[pl.ds(ebase, CH)], buf0, rs0)

    def step(i, carry):
        j0 = 2 * i
        j1 = 2 * i + 1
        pltpu.async_copy(ea_hbm.at[pl.ds(ebase + j1 * CH, CH)], buf1, rs1)
        pltpu.make_async_copy(ea_hbm.at[pl.ds(ebase, CH)], buf0, rs0).wait()
        pltpu.sync_copy(buf0, agg_s.at[idx_v.at[pl.ds(j0 * CH, CH)]],
                        add=True)
        j2 = jnp.minimum(j0 + 2, NFC - 1)
        pltpu.async_copy(ea_hbm.at[pl.ds(ebase + j2 * CH, CH)], buf0, rs0)
        pltpu.make_async_copy(ea_hbm.at[pl.ds(ebase, CH)], buf1, rs1).wait()
        pltpu.sync_copy(buf1, agg_s.at[idx_v.at[pl.ds(j1 * CH, CH)]],
                        add=True)
        return carry

    lax.fori_loop(0, NFC // 2, step, 0)
    pltpu.make_async_copy(ea_hbm.at[pl.ds(ebase, CH)], buf0, rs0).wait()
    pltpu.sync_copy(ea_hbm.at[pl.ds(ebase + NFC * CH, TAIL)],
                    buf0.at[pl.ds(0, TAIL)])
    pltpu.sync_copy(buf0, agg_s.at[idx_v.at[pl.ds(NFC * CH, CH)]], add=True)

    plsc.subcore_barrier()
    pltpu.sync_copy(agg_s.at[pl.ds(r0, ROWS_PT)],
                    out_hbm.at[cid, pl.ds(r0, ROWS_PT)])


BN = 1000


def _mlp_a_body(x_ref, w1x_ref, b1_ref, t_ref):
    t_ref[...] = (jnp.dot(x_ref[...], w1x_ref[...],
                          preferred_element_type=jnp.float32) + b1_ref[...])


def _mlp_a(x, w1x, b1):
    return pl.pallas_call(
        _mlp_a_body,
        grid=(N_NODES // BN,),
        in_specs=[
            pl.BlockSpec((BN, D), lambda i: (i, 0)),
            pl.BlockSpec((D, D), lambda i: (0, 0)),
            pl.BlockSpec((1, D), lambda i: (0, 0)),
        ],
        out_specs=pl.BlockSpec((BN, D), lambda i: (i, 0)),
        out_shape=jax.ShapeDtypeStruct((N_NODES, D), jnp.float32),
    )(x, w1x, b1)


def _mlp_b_body(t_ref, p_ref, w1a_ref, w2_ref, b2_ref, o_ref):
    agg = p_ref[0] + p_ref[1]
    h = t_ref[...] + jnp.dot(agg, w1a_ref[...],
                             preferred_element_type=jnp.float32)
    h = jnp.maximum(h, 0.0)
    o_ref[...] = (jnp.dot(h, w2_ref[...], preferred_element_type=jnp.float32)
                  + b2_ref[...])


def _mlp_b(t, parts, w1a, w2, b2):
    return pl.pallas_call(
        _mlp_b_body,
        grid=(N_NODES // BN,),
        in_specs=[
            pl.BlockSpec((BN, D), lambda i: (i, 0)),
            pl.BlockSpec((NC, BN, D), lambda i: (0, i, 0)),
            pl.BlockSpec((D, D), lambda i: (0, 0)),
            pl.BlockSpec((D, D), lambda i: (0, 0)),
            pl.BlockSpec((1, D), lambda i: (0, 0)),
        ],
        out_specs=pl.BlockSpec((BN, D), lambda i: (i, 0)),
        out_shape=jax.ShapeDtypeStruct((N_NODES, D), jnp.float32),
    )(t, parts, w1a, w2, b2)


def kernel(x, edge_index, edge_attr, u, batch, W1, b1, W2, b2):
    ridx = edge_index[0].astype(jnp.int32)
    zeros = jnp.zeros((ROWS_PT, D), jnp.float32)
    parts = _scatter_sc(ridx, edge_attr, zeros)
    w1T = W1.T
    t = _mlp_a(x, w1T[:D], b1.reshape(1, D))
    return _mlp_b(t, parts, w1T[D:], W2.T, b2.reshape(1, D))


# speedup vs baseline: 7.9198x; 1.0895x over previous
import functools

import jax
import jax.numpy as jnp
from jax import lax
from jax.experimental import pallas as pl
from jax.experimental.pallas import tpu as pltpu
from jax.experimental.pallas import tpu_sc as plsc

N_NODES = 10000
N_EDGES = 320000
D = 128
NC = 2
NS = 16
NW = NC * NS
CH = 128
NCHUNK = N_EDGES // CH
CPT = NCHUNK // NW
NX = NCHUNK - CPT * NW
EPT = CPT * CH
IDXW = EPT + CH
N_PAD = 10240
ROWS_PT = N_PAD // NS

_mesh = plsc.VectorSubcoreMesh(core_axis_name="c", subcore_axis_name="s")


@functools.partial(
    pl.kernel,
    out_type=jax.ShapeDtypeStruct((NC, N_PAD, D), jnp.float32),
    mesh=_mesh,
    scratch_types=[
        pltpu.VMEM((IDXW,), jnp.int32),
        pltpu.VMEM((CH, D), jnp.float32),
        pltpu.VMEM((CH, D), jnp.float32),
        pltpu.VMEM_SHARED((N_PAD, D), jnp.float32),
        pltpu.SemaphoreType.DMA,
        pltpu.SemaphoreType.DMA,
    ],
)
def _scatter_sc(ei_hbm, ea_hbm, zeros_hbm, out_hbm,
                idx_v, buf0, buf1, agg_s, rs0, rs1):
    cid = lax.axis_index("c")
    sid = lax.axis_index("s")
    wid = sid * NC + cid
    r0 = sid * ROWS_PT
    ebase = wid * EPT

    pltpu.async_copy(ei_hbm.at[0, pl.ds(ebase, EPT)],
                     idx_v.at[pl.ds(0, EPT)], rs1)
    pltpu.sync_copy(zeros_hbm, agg_s.at[pl.ds(r0, ROWS_PT)])
    pltpu.make_async_copy(ei_hbm.at[0, pl.ds(ebase, EPT)],
                          idx_v.at[pl.ds(0, EPT)], rs1).wait()

    @pl.when(wid < NX)
    def _():
        pltpu.sync_copy(ei_hbm.at[0, pl.ds((NW * CPT + wid) * CH, CH)],
                        idx_v.at[pl.ds(EPT, CH)])

    plsc.subcore_barrier()

    pltpu.async_copy(ea_hbm.at[pl.ds(ebase, CH)], buf0, rs0)

    def step(i, carry):
        j0 = 2 * i
        j1 = 2 * i + 1
        pltpu.async_copy(ea_hbm.at[pl.ds(ebase + j1 * CH, CH)], buf1, rs1)
        pltpu.make_async_copy(ea_hbm.at[pl.ds(ebase, CH)], buf0, rs0).wait()
        pltpu.sync_copy(buf0, agg_s.at[idx_v.at[pl.ds(j0 * CH, CH)]],
                        add=True)
        j2 = jnp.minimum(j0 + 2, CPT - 1)
        pltpu.async_copy(ea_hbm.at[pl.ds(ebase + j2 * CH, CH)], buf0, rs0)
        pltpu.make_async_copy(ea_hbm.at[pl.ds(ebase, CH)], buf1, rs1).wait()
        pltpu.sync_copy(buf1, agg_s.at[idx_v.at[pl.ds(j1 * CH, CH)]],
                        add=True)
        return carry

    lax.fori_loop(0, CPT // 2, step, 0)
    pltpu.make_async_copy(ea_hbm.at[pl.ds(ebase, CH)], buf0, rs0).wait()

    @pl.when(wid < NX)
    def _():
        pltpu.sync_copy(ea_hbm.at[pl.ds((NW * CPT + wid) * CH, CH)], buf0)
        pltpu.sync_copy(buf0, agg_s.at[idx_v.at[pl.ds(EPT, CH)]], add=True)

    plsc.subcore_barrier()
    pltpu.sync_copy(agg_s.at[pl.ds(r0, ROWS_PT)],
                    out_hbm.at[cid, pl.ds(r0, ROWS_PT)])


BN = 1000


def _mlp_a_body(x_ref, w1x_ref, b1_ref, t_ref):
    t_ref[...] = (jnp.dot(x_ref[...], w1x_ref[...],
                          preferred_element_type=jnp.float32) + b1_ref[...])


def _mlp_a(x, w1x, b1):
    return pl.pallas_call(
        _mlp_a_body,
        grid=(N_NODES // BN,),
        in_specs=[
            pl.BlockSpec((BN, D), lambda i: (i, 0)),
            pl.BlockSpec((D, D), lambda i: (0, 0)),
            pl.BlockSpec((1, D), lambda i: (0, 0)),
        ],
        out_specs=pl.BlockSpec((BN, D), lambda i: (i, 0)),
        out_shape=jax.ShapeDtypeStruct((N_NODES, D), jnp.float32),
    )(x, w1x, b1)


def _mlp_b_body(t_ref, p_ref, w1a_ref, w2_ref, b2_ref, o_ref):
    agg = p_ref[0] + p_ref[1]
    h = t_ref[...] + jnp.dot(agg, w1a_ref[...],
                             preferred_element_type=jnp.float32)
    h = jnp.maximum(h, 0.0)
    o_ref[...] = (jnp.dot(h, w2_ref[...], preferred_element_type=jnp.float32)
                  + b2_ref[...])


def _mlp_b(t, parts, w1a, w2, b2):
    return pl.pallas_call(
        _mlp_b_body,
        grid=(N_NODES // BN,),
        in_specs=[
            pl.BlockSpec((BN, D), lambda i: (i, 0)),
            pl.BlockSpec((NC, BN, D), lambda i: (0, i, 0)),
            pl.BlockSpec((D, D), lambda i: (0, 0)),
            pl.BlockSpec((D, D), lambda i: (0, 0)),
            pl.BlockSpec((1, D), lambda i: (0, 0)),
        ],
        out_specs=pl.BlockSpec((BN, D), lambda i: (i, 0)),
        out_shape=jax.ShapeDtypeStruct((N_NODES, D), jnp.float32),
    )(t, parts, w1a, w2, b2)


def kernel(x, edge_index, edge_attr, u, batch, W1, b1, W2, b2):
    zeros = jnp.zeros((ROWS_PT, D), jnp.float32)
    parts = _scatter_sc(edge_index.astype(jnp.int32), edge_attr, zeros)
    w1T = W1.T
    t = _mlp_a(x, w1T[:D], b1.reshape(1, D))
    return _mlp_b(t, parts, w1T[D:], W2.T, b2.reshape(1, D))
